# baseline (speedup 1.0000x reference)
import functools

import jax
import jax.numpy as jnp
from jax import lax
from jax.experimental import pallas as pl
from jax.experimental.pallas import tpu as pltpu
from jax.experimental.pallas import tpu_sc as plsc

_NUM_FIELDS = 26
_VOCAB = 100000
_EMB_DIM = 32
_BATCH = 16384

_NC = 2
_NS = 16
_NW = _NC * _NS
_LANES = 16
_PACK = 128 // _EMB_DIM
_BLK = 128
_BPW = _BATCH // (_NW * _BLK)
_RPW = _BPW * _BLK


@functools.partial(
    pl.kernel,
    out_type=jax.ShapeDtypeStruct((_NUM_FIELDS, _BATCH, _EMB_DIM), jnp.float32),
    mesh=plsc.VectorSubcoreMesh(core_axis_name="c", subcore_axis_name="s"),
    compiler_params=pltpu.CompilerParams(
        use_tc_tiling_on_sc=False, needs_layout_passes=False
    ),
    scratch_types=[
        pltpu.VMEM((_BPW, _BLK), jnp.int32),
        pltpu.VMEM((_BPW, _BLK), jnp.int32),
        pltpu.VMEM((_RPW, _BLK), jnp.float32),
        pltpu.VMEM((_RPW, _EMB_DIM), jnp.float32),
        pltpu.SemaphoreType.DMA,
    ],
)
def _mce_gather(idx_hbm, tab_hbm, out_hbm, idx_v, pidx_v, packed_v, rows_v, sem):
    w = lax.axis_index("s") * _NC + lax.axis_index("c")
    lane = lax.iota(jnp.int32, _LANES)

    def field_body(f, carry):
        pltpu.sync_copy(idx_hbm.at[f, w], idx_v)

        def pidx_body(g, c2):
            k = g // (_BLK // _LANES)
            t = g % (_BLK // _LANES)
            sl = pl.ds(t * _LANES, _LANES)
            pidx_v[k, sl] = lax.shift_right_logical(idx_v[k, sl], 2)
            return c2

        lax.fori_loop(0, _BPW * (_BLK // _LANES), pidx_body, 0)

        tab_f = tab_hbm.at[f]
        copies = [
            pltpu.async_copy(
                tab_f.at[pidx_v.at[k]],
                packed_v.at[pl.ds(k * _BLK, _BLK)],
                sem,
            )
            for k in range(_BPW)
        ]
        for cp in copies:
            cp.wait()

        def extract_body(g, c2):
            k = g // (_BLK // _LANES)
            t = g % (_BLK // _LANES)
            sl = pl.ds(t * _LANES, _LANES)
            rows16 = g * _LANES + lane
            sub = lax.rem(idx_v[k, sl], jnp.int32(_PACK))
            col0 = sub * _EMB_DIM
            for d in range(_EMB_DIM):
                vals = plsc.load_gather(packed_v, [rows16, col0 + d])
                plsc.store_scatter(rows_v, [rows16, jnp.full((_LANES,), d, jnp.int32)], vals)
            return c2

        lax.fori_loop(0, _RPW // _LANES, extract_body, 0)

        pltpu.sync_copy(rows_v, out_hbm.at[f, pl.ds(w * _RPW, _RPW)])
        return carry

    lax.fori_loop(0, _NUM_FIELDS, field_body, 0)


def kernel(inputs, tables):
    idx = inputs.astype(jnp.int32).T.reshape(_NUM_FIELDS, _NW, _BPW, _BLK)
    tab = tables.reshape(_NUM_FIELDS, _VOCAB // _PACK, _EMB_DIM * _PACK)
    out = _mce_gather(idx, tab)
    return out.reshape(_NUM_FIELDS, _BATCH, 1, _EMB_DIM)


# speedup vs baseline: 2.5083x; 2.5083x over previous
import functools

import jax
import jax.numpy as jnp
from jax import lax
from jax.experimental import pallas as pl
from jax.experimental.pallas import tpu as pltpu
from jax.experimental.pallas import tpu_sc as plsc

_NUM_FIELDS = 26
_VOCAB = 100000
_EMB_DIM = 32
_BATCH = 16384

_NC = 2
_NS = 16
_NW = _NC * _NS
_LANES = 16
_BLK = 128
_NCH = 2
_CBLK = _BATCH // (_NCH * _BLK)
_UNROLL = 8


@functools.partial(
    pl.kernel,
    out_type=jax.ShapeDtypeStruct(
        (_NUM_FIELDS, _EMB_DIM // 8, _BATCH // _BLK, 8, _BLK), jnp.float32
    ),
    mesh=plsc.VectorSubcoreMesh(core_axis_name="c", subcore_axis_name="s"),
    compiler_params=pltpu.CompilerParams(
        use_tc_tiling_on_sc=False, needs_layout_passes=False
    ),
    scratch_types=[
        pltpu.VMEM((_VOCAB,), jnp.float32),
        pltpu.VMEM((_CBLK, _BLK), jnp.int32),
        pltpu.VMEM((_CBLK, _BLK), jnp.float32),
    ],
)
def _mce_gather(idx_hbm, tab_hbm, out_hbm, row_v, idx_v, val_v):
    w = lax.axis_index("s") * _NC + lax.axis_index("c")
    tr = w // 8
    dr = w % 8

    def field_body(f, carry):
        pltpu.sync_copy(tab_hbm.at[f, w], row_v)

        def chunk_body(c, c2):
            pltpu.sync_copy(idx_hbm.at[f, c], idx_v)

            def burst_body(gg, c3):
                for j in range(_UNROLL):
                    g = gg * _UNROLL + j
                    k = g // (_BLK // _LANES)
                    t = g % (_BLK // _LANES)
                    sl = pl.ds(t * _LANES, _LANES)
                    vals = plsc.load_gather(row_v, [idx_v[k, sl]])
                    val_v[k, sl] = vals
                return c3

            n_groups = _CBLK * _BLK // _LANES
            lax.fori_loop(0, n_groups // _UNROLL, burst_body, 0)
            pltpu.sync_copy(
                val_v, out_hbm.at[f, tr, pl.ds(c * _CBLK, _CBLK), dr]
            )
            return c2

        lax.fori_loop(0, _NCH, chunk_body, 0)
        return carry

    lax.fori_loop(0, _NUM_FIELDS, field_body, 0)


def kernel(inputs, tables):
    idx = inputs.astype(jnp.int32).T.reshape(_NUM_FIELDS, _NCH, _CBLK, _BLK)
    tt = jnp.swapaxes(tables, 1, 2)
    out5 = _mce_gather(idx, tt)
    out = out5.transpose(0, 2, 4, 1, 3).reshape(_NUM_FIELDS, _BATCH, _EMB_DIM)
    return out.reshape(_NUM_FIELDS, _BATCH, 1, _EMB_DIM)


# speedup vs baseline: 5.9814x; 2.3846x over previous
import functools

import jax
import jax.numpy as jnp
from jax import lax
from jax.experimental import pallas as pl
from jax.experimental.pallas import tpu as pltpu
from jax.experimental.pallas import tpu_sc as plsc

_NUM_FIELDS = 26
_VOCAB = 100000
_EMB_DIM = 32
_BATCH = 16384

_NC = 2
_NS = 16
_NW = _NC * _NS
_LANES = 16
_BLK = 128
_NCH = 2
_CBLK = _BATCH // (_NCH * _BLK)
_UNROLL = 8


@functools.partial(
    pl.kernel,
    out_type=jax.ShapeDtypeStruct(
        (_NUM_FIELDS, _EMB_DIM // 8, _BATCH // _BLK, 8, _BLK), jnp.float32
    ),
    mesh=plsc.VectorSubcoreMesh(core_axis_name="c", subcore_axis_name="s"),
    compiler_params=pltpu.CompilerParams(needs_layout_passes=False),
    scratch_types=[
        pltpu.VMEM((_VOCAB,), jnp.float32),
        pltpu.VMEM((_CBLK, _BLK), jnp.int32),
        pltpu.VMEM((_CBLK, _BLK), jnp.float32),
    ],
)
def _mce_gather(idx_hbm, tab_hbm, out_hbm, row_v, idx_v, val_v):
    w = lax.axis_index("s") * _NC + lax.axis_index("c")
    tr = w // 8
    dr = w % 8

    def field_body(f, carry):
        pltpu.sync_copy(tab_hbm.at[f, w], row_v)

        def chunk_body(c, c2):
            pltpu.sync_copy(idx_hbm.at[f, c], idx_v)

            def burst_body(gg, c3):
                for j in range(_UNROLL):
                    g = gg * _UNROLL + j
                    k = g // (_BLK // _LANES)
                    t = g % (_BLK // _LANES)
                    sl = pl.ds(t * _LANES, _LANES)
                    vals = plsc.load_gather(row_v, [idx_v[k, sl]])
                    val_v[k, sl] = vals
                return c3

            n_groups = _CBLK * _BLK // _LANES
            lax.fori_loop(0, n_groups // _UNROLL, burst_body, 0)
            pltpu.sync_copy(
                val_v, out_hbm.at[f, tr, pl.ds(c * _CBLK, _CBLK), dr]
            )
            return c2

        lax.fori_loop(0, _NCH, chunk_body, 0)
        return carry

    lax.fori_loop(0, _NUM_FIELDS, field_body, 0)


def kernel(inputs, tables):
    idx = inputs.astype(jnp.int32).T.reshape(_NUM_FIELDS, _NCH, _CBLK, _BLK)
    tt = jnp.swapaxes(tables, 1, 2)
    out5 = _mce_gather(idx, tt)
    out = out5.transpose(0, 2, 4, 1, 3).reshape(_NUM_FIELDS, _BATCH, _EMB_DIM)
    return out.reshape(_NUM_FIELDS, _BATCH, 1, _EMB_DIM)


# speedup vs baseline: 6.1581x; 1.0295x over previous
import functools

import jax
import jax.numpy as jnp
from jax import lax
from jax.experimental import pallas as pl
from jax.experimental.pallas import tpu as pltpu
from jax.experimental.pallas import tpu_sc as plsc

_NUM_FIELDS = 26
_VOCAB = 100000
_EMB_DIM = 32
_BATCH = 16384

_NC = 2
_NS = 16
_NW = _NC * _NS
_LANES = 16
_BLK = 128
_NCH = 2
_CBLK = _BATCH // (_NCH * _BLK)
_UNROLL = 8


@functools.partial(
    pl.kernel,
    out_type=jax.ShapeDtypeStruct(
        (_NUM_FIELDS, _EMB_DIM // 8, _BATCH // _BLK, 8, _BLK), jnp.float32
    ),
    mesh=plsc.VectorSubcoreMesh(core_axis_name="c", subcore_axis_name="s"),
    compiler_params=pltpu.CompilerParams(needs_layout_passes=False),
    scratch_types=[
        pltpu.VMEM((_VOCAB,), jnp.float32),
        pltpu.VMEM((_CBLK, _BLK), jnp.int32),
        pltpu.VMEM((_CBLK, _BLK), jnp.float32),
        pltpu.SemaphoreType.DMA,
    ],
)
def _mce_gather(idx_hbm, tab_hbm, out_hbm, row_v, idx_v, val_v, row_sem):
    w = lax.axis_index("s") * _NC + lax.axis_index("c")
    tr = w // 8
    dr = w % 8

    def field_body(f, carry):
        row_cp = pltpu.async_copy(tab_hbm.at[f, w], row_v, row_sem)
        pltpu.sync_copy(idx_hbm.at[f, 0], idx_v)
        row_cp.wait()

        def chunk_body(c, c2):
            @pl.when(c > 0)
            def _():
                pltpu.sync_copy(idx_hbm.at[f, c], idx_v)

            def burst_body(k, c3):
                for j in range(_BLK // _LANES):
                    sl = pl.ds(j * _LANES, _LANES)
                    vals = plsc.load_gather(row_v, [idx_v[k, sl]])
                    val_v[k, sl] = vals
                return c3

            lax.fori_loop(0, _CBLK, burst_body, 0)
            pltpu.sync_copy(
                val_v, out_hbm.at[f, tr, pl.ds(c * _CBLK, _CBLK), dr]
            )
            return c2

        lax.fori_loop(0, _NCH, chunk_body, 0)
        return carry

    lax.fori_loop(0, _NUM_FIELDS, field_body, 0)


def kernel(inputs, tables):
    idx = inputs.astype(jnp.int32).T.reshape(_NUM_FIELDS, _NCH, _CBLK, _BLK)
    tt = jnp.swapaxes(tables, 1, 2)
    out5 = _mce_gather(idx, tt)
    out = out5.transpose(0, 2, 4, 1, 3).reshape(_NUM_FIELDS, _BATCH, _EMB_DIM)
    return out.reshape(_NUM_FIELDS, _BATCH, 1, _EMB_DIM)


# speedup vs baseline: 7.9259x; 1.2871x over previous
import functools

import jax
import jax.numpy as jnp
from jax import lax
from jax.experimental import pallas as pl
from jax.experimental.pallas import tpu as pltpu
from jax.experimental.pallas import tpu_sc as plsc

_NUM_FIELDS = 26
_VOCAB = 100000
_EMB_DIM = 32
_BATCH = 16384

_NC = 2
_NS = 16
_NW = _NC * _NS
_LANES = 16
_BLK = 128
_NCH = 2
_CBLK = _BATCH // (_NCH * _BLK)
_UNROLL = 8


@functools.partial(
    pl.kernel,
    out_type=jax.ShapeDtypeStruct(
        (_NUM_FIELDS, _EMB_DIM // 8, _BATCH // _BLK, 8, _BLK), jnp.float32
    ),
    mesh=plsc.VectorSubcoreMesh(core_axis_name="c", subcore_axis_name="s"),
    compiler_params=pltpu.CompilerParams(needs_layout_passes=False),
    scratch_types=[
        pltpu.VMEM((_VOCAB,), jnp.float32),
        pltpu.VMEM((_CBLK, _BLK), jnp.int32),
        pltpu.VMEM((_CBLK, _BLK), jnp.float32),
        pltpu.SemaphoreType.DMA,
    ],
)
def _mce_gather(idx_hbm, tab_hbm, out_hbm, row_v, idx_v, val_v, row_sem):
    w = lax.axis_index("s") * _NC + lax.axis_index("c")
    tr = w // 8
    dr = w % 8

    def field_body(f, carry):
        row_cp = pltpu.async_copy(tab_hbm.at[f, w], row_v, row_sem)
        pltpu.sync_copy(idx_hbm.at[f, 0], idx_v)
        row_cp.wait()

        def chunk_body(c, c2):
            @pl.when(c > 0)
            def _():
                pltpu.sync_copy(idx_hbm.at[f, c], idx_v)

            @plsc.parallel_loop(0, _CBLK, unroll=2)
            def _(k):
                for j in range(_BLK // _LANES):
                    sl = pl.ds(j * _LANES, _LANES)
                    vals = plsc.load_gather(row_v, [idx_v[k, sl]])
                    val_v[k, sl] = vals
            pltpu.sync_copy(
                val_v, out_hbm.at[f, tr, pl.ds(c * _CBLK, _CBLK), dr]
            )
            return c2

        lax.fori_loop(0, _NCH, chunk_body, 0)
        return carry

    lax.fori_loop(0, _NUM_FIELDS, field_body, 0)


def kernel(inputs, tables):
    idx = inputs.astype(jnp.int32).T.reshape(_NUM_FIELDS, _NCH, _CBLK, _BLK)
    tt = jnp.swapaxes(tables, 1, 2)
    out5 = _mce_gather(idx, tt)
    out = out5.transpose(0, 2, 4, 1, 3).reshape(_NUM_FIELDS, _BATCH, _EMB_DIM)
    return out.reshape(_NUM_FIELDS, _BATCH, 1, _EMB_DIM)


# speedup vs baseline: 8.4894x; 1.0711x over previous
import functools

import jax
import jax.numpy as jnp
from jax import lax
from jax.experimental import pallas as pl
from jax.experimental.pallas import tpu as pltpu
from jax.experimental.pallas import tpu_sc as plsc

_NUM_FIELDS = 26
_VOCAB = 100000
_EMB_DIM = 32
_BATCH = 16384

_NC = 2
_NS = 16
_NW = _NC * _NS
_LANES = 16
_BLK = 128
_NCH = 2
_CBLK = _BATCH // (_NCH * _BLK)


@functools.partial(
    pl.kernel,
    out_type=jax.ShapeDtypeStruct(
        (_NUM_FIELDS, _EMB_DIM // 8, _BATCH // _BLK, 8, _BLK), jnp.float32
    ),
    mesh=plsc.VectorSubcoreMesh(core_axis_name="c", subcore_axis_name="s"),
    compiler_params=pltpu.CompilerParams(needs_layout_passes=False),
    scratch_types=[
        pltpu.VMEM((_VOCAB,), jnp.float32),
        pltpu.VMEM((_CBLK, _BLK), jnp.int32),
        pltpu.VMEM((_CBLK, _BLK), jnp.float32),
        pltpu.VMEM((_CBLK, _BLK), jnp.float32),
        pltpu.SemaphoreType.DMA,
        pltpu.SemaphoreType.DMA,
        pltpu.SemaphoreType.DMA,
    ],
)
def _mce_gather(
    idx_hbm, tab_hbm, out_hbm, row_v, idx_v, val_a, val_b, row_sem, sem_a, sem_b
):
    w = lax.axis_index("s") * _NC + lax.axis_index("c")
    tr = w // 8
    dr = w % 8

    def out_slice(f, c):
        return out_hbm.at[f, tr, pl.ds(c * _CBLK, _CBLK), dr]

    def field_body(f, carry):
        row_cp = pltpu.async_copy(tab_hbm.at[f, w], row_v, row_sem)
        pltpu.sync_copy(idx_hbm.at[f, 0], idx_v)
        row_cp.wait()

        for c, (val_v, osem) in enumerate(((val_a, sem_a), (val_b, sem_b))):
            if c > 0:
                pltpu.sync_copy(idx_hbm.at[f, c], idx_v)

            @pl.when(f > 0)
            def _():
                pltpu.make_async_copy(val_v, out_slice(f, c), osem).wait()

            @plsc.parallel_loop(0, _CBLK, unroll=2)
            def _(k):
                for j in range(_BLK // _LANES):
                    sl = pl.ds(j * _LANES, _LANES)
                    vals = plsc.load_gather(row_v, [idx_v[k, sl]])
                    val_v[k, sl] = vals

            pltpu.async_copy(val_v, out_slice(f, c), osem)
        return carry

    lax.fori_loop(0, _NUM_FIELDS, field_body, 0)
    pltpu.make_async_copy(val_a, out_slice(_NUM_FIELDS - 1, 0), sem_a).wait()
    pltpu.make_async_copy(val_b, out_slice(_NUM_FIELDS - 1, 1), sem_b).wait()


def kernel(inputs, tables):
    idx = inputs.astype(jnp.int32).T.reshape(_NUM_FIELDS, _NCH, _CBLK, _BLK)
    tt = jnp.swapaxes(tables, 1, 2)
    out5 = _mce_gather(idx, tt)
    out = out5.transpose(0, 2, 4, 1, 3).reshape(_NUM_FIELDS, _BATCH, _EMB_DIM)
    return out.reshape(_NUM_FIELDS, _BATCH, 1, _EMB_DIM)


# speedup vs baseline: 8.5261x; 1.0043x over previous
import functools

import jax
import jax.numpy as jnp
from jax import lax
from jax.experimental import pallas as pl
from jax.experimental.pallas import tpu as pltpu
from jax.experimental.pallas import tpu_sc as plsc

_NUM_FIELDS = 26
_VOCAB = 100000
_EMB_DIM = 32
_BATCH = 16384

_NC = 2
_NS = 16
_NW = _NC * _NS
_LANES = 16
_BLK = 128
_NCH = 2
_CBLK = _BATCH // (_NCH * _BLK)


@functools.partial(
    pl.kernel,
    out_type=jax.ShapeDtypeStruct(
        (_NUM_FIELDS, _EMB_DIM // 8, _BATCH // _BLK, 8, _BLK), jnp.float32
    ),
    mesh=plsc.VectorSubcoreMesh(core_axis_name="c", subcore_axis_name="s"),
    compiler_params=pltpu.CompilerParams(needs_layout_passes=False),
    scratch_types=[
        pltpu.VMEM((_VOCAB,), jnp.float32),
        pltpu.VMEM((_CBLK, _BLK), jnp.int32),
        pltpu.VMEM((_CBLK, _BLK), jnp.float32),
        pltpu.VMEM((_CBLK, _BLK), jnp.float32),
        pltpu.SemaphoreType.DMA,
        pltpu.SemaphoreType.DMA,
        pltpu.SemaphoreType.DMA,
    ],
)
def _mce_gather(
    idx_hbm, tab_hbm, out_hbm, row_v, idx_v, val_a, val_b, row_sem, sem_a, sem_b
):
    w = lax.axis_index("s") * _NC + lax.axis_index("c")
    tr = w // 8
    dr = w % 8

    def out_slice(f, c):
        return out_hbm.at[f, tr, pl.ds(c * _CBLK, _CBLK), dr]

    def field_body(f, carry):
        row_cp = pltpu.async_copy(tab_hbm.at[f, w], row_v, row_sem)
        pltpu.sync_copy(idx_hbm.at[f, 0], idx_v)
        row_cp.wait()

        for c, (val_v, osem) in enumerate(((val_a, sem_a), (val_b, sem_b))):
            if c > 0:
                pltpu.sync_copy(idx_hbm.at[f, c], idx_v)

            @pl.when(f > 0)
            def _():
                pltpu.make_async_copy(val_v, out_slice(f, c), osem).wait()

            @plsc.parallel_loop(0, _CBLK, unroll=4)
            def _(k):
                for j in range(_BLK // _LANES):
                    sl = pl.ds(j * _LANES, _LANES)
                    vals = plsc.load_gather(row_v, [idx_v[k, sl]])
                    val_v[k, sl] = vals

            pltpu.async_copy(val_v, out_slice(f, c), osem)
        return carry

    lax.fori_loop(0, _NUM_FIELDS, field_body, 0)
    pltpu.make_async_copy(val_a, out_slice(_NUM_FIELDS - 1, 0), sem_a).wait()
    pltpu.make_async_copy(val_b, out_slice(_NUM_FIELDS - 1, 1), sem_b).wait()


def kernel(inputs, tables):
    idx = inputs.astype(jnp.int32).T.reshape(_NUM_FIELDS, _NCH, _CBLK, _BLK)
    tt = jnp.swapaxes(tables, 1, 2)
    out5 = _mce_gather(idx, tt)
    out = out5.transpose(0, 2, 4, 1, 3).reshape(_NUM_FIELDS, _BATCH, _EMB_DIM)
    return out.reshape(_NUM_FIELDS, _BATCH, 1, _EMB_DIM)
